# baseline (speedup 1.0000x reference)
import functools

import jax
import jax.numpy as jnp
from jax import lax
from jax.experimental import pallas as pl
from jax.experimental.pallas import tpu as pltpu
from jax.experimental.pallas import tpu_sc as plsc

_B = 16
_H = 512
_W = 512
_PW = 514
_PV = _PW * _PW
_NF = _B * _PV
_N = _B * _H * _W
_IDXW = 128
_NR = _N // _IDXW
_NWORK = 32
_CH = 4
_RB = 256


def _coords(dx, dy, r0):
    ii = lax.broadcasted_iota(jnp.int32, dx.shape, 0).astype(jnp.float32)
    jj = lax.broadcasted_iota(jnp.int32, dx.shape, 1).astype(jnp.float32)
    ax = (ii + jnp.float32(r0)) * jnp.float32(2.0 / 511.0) - 1.0
    ay = jj * jnp.float32(2.0 / 511.0) - 1.0
    x = 0.5 * (ax - dx + 1.0) * 511.0
    y = 0.5 * (ay - dy + 1.0) * 511.0
    return x, y


def _idx_body(dx_ref, dy_ref, idx_ref):
    b = pl.program_id(0)
    r0 = pl.program_id(1) * _RB
    x, y = _coords(dx_ref[0], dy_ref[0], r0)
    x0 = jnp.where(x < 0, 512, jnp.minimum(jnp.floor(x).astype(jnp.int32), 512))
    y0 = jnp.where(y < 0, 512, jnp.minimum(jnp.floor(y).astype(jnp.int32), 512))
    idx_ref[0] = b * _PV + x0 * _PW + y0


def _combine_body(dx_ref, dy_ref, g00_ref, g01_ref, g10_ref, g11_ref, out_ref):
    r0 = pl.program_id(1) * _RB
    x, y = _coords(dx_ref[0], dy_ref[0], r0)
    x0 = jnp.clip(jnp.floor(x).astype(jnp.int32), 0, 512)
    y0 = jnp.clip(jnp.floor(y).astype(jnp.int32), 0, 512)
    x1 = jnp.minimum(x0 + 1, 512)
    y1 = jnp.minimum(y0 + 1, 512)
    bx = x1.astype(jnp.float32) - x
    axf = x - x0.astype(jnp.float32)
    by = y1.astype(jnp.float32) - y
    ayf = y - y0.astype(jnp.float32)
    out_ref[0] = (bx * by * g00_ref[0] + bx * ayf * g01_ref[0]
                  + axf * by * g10_ref[0] + axf * ayf * g11_ref[0])


def _row_spec():
    return pl.BlockSpec((1, _RB, _W), lambda b, r: (b, r, 0))


def _compute_indices(dx, dy):
    return pl.pallas_call(
        _idx_body,
        grid=(_B, _H // _RB),
        in_specs=[_row_spec(), _row_spec()],
        out_specs=_row_spec(),
        out_shape=jax.ShapeDtypeStruct((_B, _H, _W), jnp.int32),
    )(dx, dy)


def _combine(dx, dy, g00, g01, g10, g11):
    return pl.pallas_call(
        _combine_body,
        grid=(_B, _H // _RB),
        in_specs=[_row_spec()] * 6,
        out_specs=_row_spec(),
        out_shape=jax.ShapeDtypeStruct((_B, _H, _W), jnp.float32),
    )(dx, dy, g00, g01, g10, g11)


@functools.cache
def _make_sc_gather():
    plane = jax.ShapeDtypeStruct((_NR, _IDXW), jnp.float32)

    @functools.partial(
        pl.kernel,
        out_type=[plane, plane, plane, plane],
        mesh=plsc.VectorSubcoreMesh(core_axis_name="c", subcore_axis_name="s"),
        scratch_types=[
            pltpu.VMEM((2, _CH, _IDXW), jnp.int32),
            pltpu.VMEM((2, _CH, _IDXW), jnp.float32),
            pltpu.VMEM((2, _CH, _IDXW), jnp.float32),
            pltpu.VMEM((2, _CH, _IDXW), jnp.float32),
            pltpu.VMEM((2, _CH, _IDXW), jnp.float32),
            pltpu.SemaphoreType.DMA,
            pltpu.SemaphoreType.DMA,
            pltpu.SemaphoreType.DMA,
            pltpu.SemaphoreType.DMA,
            pltpu.SemaphoreType.DMA,
            pltpu.SemaphoreType.DMA,
        ],
        compiler_params=pltpu.CompilerParams(use_tc_tiling_on_sc=False),
    )
    def _sc_gather(f0, f1, f2, f3, idx_hbm, o0, o1, o2, o3,
                   idx_v, r0, r1, r2, r3,
                   si_a, si_b, sg_a, sg_b, sw_a, sw_b):
        wid = lax.axis_index("s") * 2 + lax.axis_index("c")
        rows_per_worker = _NR // _NWORK
        n = rows_per_worker // _CH
        base0 = wid * rows_per_worker
        srcs = (f0, f1, f2, f3)
        rbufs = (r0, r1, r2, r3)
        outs = (o0, o1, o2, o3)
        si = (si_a, si_b)
        sg = (sg_a, sg_b)
        sw = (sw_a, sw_b)

        def fire_idx(ci, par, sem):
            pltpu.async_copy(idx_hbm.at[pl.ds(base0 + ci * _CH, _CH)],
                             idx_v.at[par], sem)

        def wait_idx(par, sem):
            pltpu.make_async_copy(idx_hbm.at[pl.ds(0, _CH)],
                                  idx_v.at[par], sem).wait()

        def fire_gathers(par, sem):
            for c in range(4):
                for k in range(_CH):
                    pltpu.async_copy(srcs[c].at[idx_v.at[par, k]],
                                     rbufs[c].at[par, k], sem)

        def wait_gathers(par, sem):
            for c in range(4):
                pltpu.make_async_copy(
                    outs[c].at[pl.ds(0, _CH)], rbufs[c].at[par], sem).wait()

        def fire_wb(ci, par, sem):
            for c in range(4):
                pltpu.async_copy(rbufs[c].at[par],
                                 outs[c].at[pl.ds(base0 + ci * _CH, _CH)], sem)

        def wait_wb(par, sem):
            for c in range(4):
                pltpu.make_async_copy(
                    rbufs[c].at[par], outs[c].at[pl.ds(0, _CH)], sem).wait()

        fire_idx(0, 0, si[0])

        @pl.loop(0, n, step=2)
        def _chunk(g):
            for par in (0, 1):
                ci = g + par
                wait_idx(par, si[par])

                @pl.when(ci >= 2)
                def _():
                    wait_wb(par, sw[par])

                fire_gathers(par, sg[par])

                @pl.when(ci >= 1)
                def _():
                    wait_gathers(1 - par, sg[1 - par])
                    fire_wb(ci - 1, 1 - par, sw[1 - par])

                @pl.when(ci + 1 < n)
                def _():
                    fire_idx(ci + 1, 1 - par, si[1 - par])

        wait_gathers(1, sg[1])
        fire_wb(n - 1, 1, sw[1])
        wait_wb(0, sw[0])
        wait_wb(1, sw[1])

    return _sc_gather


def _build_shift_tables(img):
    pad = jnp.pad(img, ((0, 0), (0, 2), (0, 2)))
    flat = pad.reshape(_B * _PV)

    def shift(k):
        return jnp.concatenate([flat[k:], jnp.zeros((k,), jnp.float32)])

    return flat, shift(1), shift(_PW), shift(_PW + 1)


def kernel(Img, DispField):
    img = Img.reshape(_B, _H, _W)
    dx = DispField[..., 0]
    dy = DispField[..., 1]
    idx = _compute_indices(dx, dy)
    f0, f1, f2, f3 = _build_shift_tables(img)
    g00, g01, g10, g11 = _make_sc_gather()(f0, f1, f2, f3,
                                           idx.reshape(_NR, _IDXW))
    out = _combine(dx, dy,
                   g00.reshape(_B, _H, _W), g01.reshape(_B, _H, _W),
                   g10.reshape(_B, _H, _W), g11.reshape(_B, _H, _W))
    return out.reshape(_B, _H, _W, 1)


# speedup vs baseline: 1.0275x; 1.0275x over previous
import functools

import jax
import jax.numpy as jnp
from jax import lax
from jax.experimental import pallas as pl
from jax.experimental.pallas import tpu as pltpu
from jax.experimental.pallas import tpu_sc as plsc

_B = 16
_H = 512
_W = 512
_PW = 514
_PV = _PW * _PW
_NF = _B * _PV
_N = _B * _H * _W
_IDXW = 128
_NR = _N // _IDXW
_NWORK = 32
_CH = 4
_RB = 256


def _coords(dx, dy, r0):
    ii = lax.broadcasted_iota(jnp.int32, dx.shape, 0).astype(jnp.float32)
    jj = lax.broadcasted_iota(jnp.int32, dx.shape, 1).astype(jnp.float32)
    ax = (ii + jnp.float32(r0)) * jnp.float32(2.0 / 511.0) - 1.0
    ay = jj * jnp.float32(2.0 / 511.0) - 1.0
    x = 0.5 * (ax - dx + 1.0) * 511.0
    y = 0.5 * (ay - dy + 1.0) * 511.0
    return x, y


def _idx_body(dx_ref, dy_ref, idx_ref):
    b = pl.program_id(0)
    r0 = pl.program_id(1) * _RB
    x, y = _coords(dx_ref[0], dy_ref[0], r0)
    x0 = jnp.where(x < 0, 512, jnp.minimum(jnp.floor(x).astype(jnp.int32), 512))
    y0 = jnp.where(y < 0, 512, jnp.minimum(jnp.floor(y).astype(jnp.int32), 512))
    idx_ref[0] = b * _PV + x0 * _PW + y0


def _combine_body(dx_ref, dy_ref, g00_ref, g01_ref, g10_ref, g11_ref, out_ref):
    r0 = pl.program_id(1) * _RB
    x, y = _coords(dx_ref[0], dy_ref[0], r0)
    x0 = jnp.clip(jnp.floor(x).astype(jnp.int32), 0, 512)
    y0 = jnp.clip(jnp.floor(y).astype(jnp.int32), 0, 512)
    x1 = jnp.minimum(x0 + 1, 512)
    y1 = jnp.minimum(y0 + 1, 512)
    bx = x1.astype(jnp.float32) - x
    axf = x - x0.astype(jnp.float32)
    by = y1.astype(jnp.float32) - y
    ayf = y - y0.astype(jnp.float32)
    out_ref[0] = (bx * by * g00_ref[0] + bx * ayf * g01_ref[0]
                  + axf * by * g10_ref[0] + axf * ayf * g11_ref[0])


def _row_spec():
    return pl.BlockSpec((1, _RB, _W), lambda b, r: (b, r, 0))


def _compute_indices(dx, dy):
    return pl.pallas_call(
        _idx_body,
        grid=(_B, _H // _RB),
        in_specs=[_row_spec(), _row_spec()],
        out_specs=_row_spec(),
        out_shape=jax.ShapeDtypeStruct((_B, _H, _W), jnp.int32),
    )(dx, dy)


def _combine(dx, dy, g00, g01, g10, g11):
    return pl.pallas_call(
        _combine_body,
        grid=(_B, _H // _RB),
        in_specs=[_row_spec()] * 6,
        out_specs=_row_spec(),
        out_shape=jax.ShapeDtypeStruct((_B, _H, _W), jnp.float32),
    )(dx, dy, g00, g01, g10, g11)


@functools.cache
def _make_sc_gather():
    plane = jax.ShapeDtypeStruct((_NR, _IDXW), jnp.float32)

    @functools.partial(
        pl.kernel,
        out_type=[plane, plane, plane, plane],
        mesh=plsc.VectorSubcoreMesh(core_axis_name="c", subcore_axis_name="s"),
        scratch_types=[
            pltpu.VMEM((_CH, _IDXW), jnp.int32),
            pltpu.VMEM((_CH, _IDXW), jnp.float32),
            pltpu.VMEM((_CH, _IDXW), jnp.float32),
            pltpu.VMEM((_CH, _IDXW), jnp.float32),
            pltpu.VMEM((_CH, _IDXW), jnp.float32),
            pltpu.SemaphoreType.DMA,
        ],
        compiler_params=pltpu.CompilerParams(use_tc_tiling_on_sc=False),
    )
    def _sc_gather(f0, f1, f2, f3, idx_hbm, o0, o1, o2, o3,
                   idx_v, r0, r1, r2, r3, sem):
        wid = lax.axis_index("s") * 2 + lax.axis_index("c")
        rows_per_worker = _NR // _NWORK
        base0 = wid * rows_per_worker
        srcs = (f0, f1, f2, f3)
        dsts = (r0, r1, r2, r3)
        outs = (o0, o1, o2, o3)

        @pl.loop(0, rows_per_worker // _CH)
        def _chunk(ci):
            base = base0 + ci * _CH
            pltpu.sync_copy(idx_hbm.at[pl.ds(base, _CH)], idx_v)
            copies = [
                pltpu.async_copy(srcs[c].at[idx_v.at[k]], dsts[c].at[k], sem)
                for k in range(_CH) for c in range(4)
            ]
            for cp in copies:
                cp.wait()
            for c in range(4):
                pltpu.sync_copy(dsts[c], outs[c].at[pl.ds(base, _CH)])

    return _sc_gather


def _build_shift_tables(img):
    pad = jnp.pad(img, ((0, 0), (0, 2), (0, 2)))
    flat = pad.reshape(_B * _PV)

    def shift(k):
        return jnp.concatenate([flat[k:], jnp.zeros((k,), jnp.float32)])

    return flat, shift(1), shift(_PW), shift(_PW + 1)


def kernel(Img, DispField):
    img = Img.reshape(_B, _H, _W)
    dx = DispField[..., 0]
    dy = DispField[..., 1]
    idx = _compute_indices(dx, dy)
    f0, f1, f2, f3 = _build_shift_tables(img)
    g00, g01, g10, g11 = _make_sc_gather()(f0, f1, f2, f3,
                                           idx.reshape(_NR, _IDXW))
    out = _combine(dx, dy,
                   g00.reshape(_B, _H, _W), g01.reshape(_B, _H, _W),
                   g10.reshape(_B, _H, _W), g11.reshape(_B, _H, _W))
    return out.reshape(_B, _H, _W, 1)
